# baseline (speedup 1.0000x reference)
import functools

import jax
import jax.numpy as jnp
from jax import lax
from jax.experimental import pallas as pl
from jax.experimental.pallas import tpu as pltpu
from jax.experimental.pallas import tpu_sc as plsc

N = 10000
D = 512
BOT = 64
H = 256
C = 46
K = 10

BR = 1000
BC = 2048
NCP = 10240
NEG = -1e30
BIGF = 1e9
KS = 16


def _adapter_body(alpha_ref, x_ref, w1_ref, b1_ref, w2_ref, b2_ref, gw1_ref,
                  emb_ref, ew1_ref):
    x = x_ref[...]
    t = jnp.maximum(jnp.dot(x, w1_ref[...], preferred_element_type=jnp.float32)
                    + b1_ref[...], 0.0)
    a2 = jnp.dot(t, w2_ref[...], preferred_element_type=jnp.float32) + b2_ref[...]
    alpha = alpha_ref[0]
    ad = alpha * (a2 + x) + (1.0 - alpha) * x
    nrm = jnp.maximum(jnp.sqrt(jnp.sum(ad * ad, axis=1, keepdims=True)), 1e-12)
    emb = ad / nrm
    emb_ref[...] = emb
    ew1_ref[...] = jnp.dot(emb, gw1_ref[...], preferred_element_type=jnp.float32)


def _adapter_call(x, W1, b1, W2, b2, alpha, gW1):
    grid = (N // BR,)
    return pl.pallas_call(
        _adapter_body,
        grid_spec=pltpu.PrefetchScalarGridSpec(
            num_scalar_prefetch=1,
            grid=grid,
            in_specs=[
                pl.BlockSpec((BR, D), lambda i, a: (i, 0)),
                pl.BlockSpec((D, BOT), lambda i, a: (0, 0)),
                pl.BlockSpec((BOT,), lambda i, a: (0,)),
                pl.BlockSpec((BOT, D), lambda i, a: (0, 0)),
                pl.BlockSpec((D,), lambda i, a: (0,)),
                pl.BlockSpec((D, H), lambda i, a: (0, 0)),
            ],
            out_specs=[
                pl.BlockSpec((BR, D), lambda i, a: (i, 0)),
                pl.BlockSpec((BR, H), lambda i, a: (i, 0)),
            ],
        ),
        out_shape=[
            jax.ShapeDtypeStruct((N, D), jnp.float32),
            jax.ShapeDtypeStruct((N, H), jnp.float32),
        ],
        compiler_params=pltpu.CompilerParams(
            dimension_semantics=("parallel",)),
    )(jnp.reshape(alpha, (1,)), x, W1, b1, W2, b2, gW1)


def _topk_extract(s, gidx, k):
    vs, ids = [], []
    for _ in range(k):
        m = jnp.max(s, axis=1, keepdims=True)
        cand = jnp.where(s >= m, gidx, BIGF)
        am = jnp.min(cand, axis=1, keepdims=True)
        vs.append(m)
        ids.append(am)
        s = jnp.where(cand == am, NEG, s)
    return jnp.concatenate(vs, axis=1), jnp.concatenate(ids, axis=1)


def _simtopk_body(rows_ref, cols_ref, oidx_ref, ow_ref, odinv_ref,
                  sv_ref, si_ref):
    i = pl.program_id(0)
    j = pl.program_id(1)
    nj = pl.num_programs(1)

    s = lax.dot_general(rows_ref[...], cols_ref[...],
                        (((1,), (1,)), ((), ())),
                        preferred_element_type=jnp.float32)
    gcol = (lax.broadcasted_iota(jnp.int32, (BR, BC), 1).astype(jnp.float32)
            + (j * BC).astype(jnp.float32))
    grow = (lax.broadcasted_iota(jnp.int32, (BR, 1), 0).astype(jnp.float32)
            + (i * BR).astype(jnp.float32))
    s = jnp.where(gcol == grow, NEG, s)
    s = jnp.where(gcol >= N, NEG, s)

    tv, ti = _topk_extract(s, gcol, K)

    @pl.when(j == 0)
    def _init():
        sv_ref[...] = jnp.full((BR, KS), NEG, jnp.float32)
        si_ref[...] = jnp.full((BR, KS), -1.0, jnp.float32)

    mv, mi = _topk_extract(jnp.concatenate([sv_ref[...], tv], axis=1),
                           jnp.concatenate([si_ref[...], ti], axis=1), K)
    pad_v = jnp.full((BR, KS - K), NEG, jnp.float32)
    pad_i = jnp.full((BR, KS - K), -1.0, jnp.float32)
    sv_ref[...] = jnp.concatenate([mv, pad_v], axis=1)
    si_ref[...] = jnp.concatenate([mi, pad_i], axis=1)

    @pl.when(j == nj - 1)
    def _emit():
        deg = 1.0 + jnp.sum(mv, axis=1, keepdims=True)
        dinv = jnp.where(deg > 0, lax.rsqrt(jnp.abs(deg) + 1e-30), 0.0)
        growb = jnp.broadcast_to(grow, (BR, KS - K))
        oidx_ref[...] = jnp.concatenate([mi, growb], axis=1).astype(jnp.int32)
        ow_ref[...] = jnp.concatenate(
            [dinv * mv, dinv, jnp.zeros((BR, KS - K - 1), jnp.float32)], axis=1)
        odinv_ref[...] = jnp.broadcast_to(dinv, (BR, 8))


def _simtopk_call(emb, embp):
    grid = (N // BR, NCP // BC)
    return pl.pallas_call(
        _simtopk_body,
        grid=grid,
        in_specs=[
            pl.BlockSpec((BR, D), lambda i, j: (i, 0)),
            pl.BlockSpec((BC, D), lambda i, j: (j, 0)),
        ],
        out_specs=[
            pl.BlockSpec((BR, KS), lambda i, j: (i, 0)),
            pl.BlockSpec((BR, KS), lambda i, j: (i, 0)),
            pl.BlockSpec((BR, 8), lambda i, j: (i, 0)),
        ],
        out_shape=[
            jax.ShapeDtypeStruct((N, KS), jnp.int32),
            jax.ShapeDtypeStruct((N, KS), jnp.float32),
            jax.ShapeDtypeStruct((N, 8), jnp.float32),
        ],
        scratch_shapes=[
            pltpu.VMEM((BR, KS), jnp.float32),
            pltpu.VMEM((BR, KS), jnp.float32),
        ],
        compiler_params=pltpu.CompilerParams(
            dimension_semantics=("parallel", "arbitrary")),
    )(emb, embp)


def _mlp2_body(msum_ref, gb1_ref, gw2_ref, hw2_ref):
    h = jnp.maximum(msum_ref[...] + gb1_ref[...], 0.0)
    hw2_ref[...] = jnp.dot(h, gw2_ref[...], preferred_element_type=jnp.float32)


def _mlp2_call(msum1, gb1, gW2p, cp):
    grid = (N // BR,)
    return pl.pallas_call(
        _mlp2_body,
        grid=grid,
        in_specs=[
            pl.BlockSpec((BR, H), lambda i: (i, 0)),
            pl.BlockSpec((H,), lambda i: (0,)),
            pl.BlockSpec((H, cp), lambda i: (0, 0)),
        ],
        out_specs=pl.BlockSpec((BR, cp), lambda i: (i, 0)),
        out_shape=jax.ShapeDtypeStruct((N, cp), jnp.float32),
        compiler_params=pltpu.CompilerParams(
            dimension_semantics=("parallel",)),
    )(msum1, gb1, gW2p)


NSC = 10240
RT = 320
GN = 8
NG = RT // GN
SKS = 11
EDG = GN * SKS


def _sc_mp_call(hw, idxf, wf, bias, dh):
    nv = dh // 16
    mesh = plsc.VectorSubcoreMesh(core_axis_name="c", subcore_axis_name="s")

    def body(hw_hbm, idx_hbm, wx_hbm, bias_hbm, out_hbm,
             idx_v, wxa, wxb, bufa, bufb, outc, bias_v, sema, semb):
        wid = lax.axis_index("s") * 2 + lax.axis_index("c")
        base = wid * RT
        pltpu.sync_copy(idx_hbm.at[pl.ds(base * SKS, RT * SKS)], idx_v)
        pltpu.sync_copy(bias_hbm, bias_v)

        def start(g, buf, wx, sem):
            gg = g % NG
            pltpu.async_copy(
                hw_hbm.at[idx_v.at[pl.ds(gg * EDG, EDG)]], buf, sem)
            pltpu.async_copy(
                wx_hbm.at[pl.ds((base * SKS + gg * EDG) * 16, EDG * 16)],
                wx, sem)

        def wait(buf, wx, sem):
            pltpu.make_async_copy(
                hw_hbm.at[idx_v.at[pl.ds(0, EDG)]], buf, sem).wait()
            pltpu.make_async_copy(
                wx_hbm.at[pl.ds(0, EDG * 16)], wx, sem).wait()

        def compute(g, buf, wx):
            for n in range(GN):
                acc = [bias_v[pl.ds(v * 16, 16)] for v in range(nv)]
                for e in range(SKS):
                    row = n * SKS + e
                    wv = wx[pl.ds(row * 16, 16)]
                    for v in range(nv):
                        acc[v] = acc[v] + wv * buf[row, pl.ds(v * 16, 16)]
                for v in range(nv):
                    outc[n, pl.ds(v * 16, 16)] = acc[v]
            pltpu.sync_copy(outc, out_hbm.at[pl.ds(base + g * GN, GN)])

        start(0, bufa, wxa, sema)
        start(1, bufb, wxb, semb)

        def pair(p, carry):
            g = p * 2
            wait(bufa, wxa, sema)
            compute(g, bufa, wxa)
            start(g + 2, bufa, wxa, sema)
            wait(bufb, wxb, semb)
            compute(g + 1, bufb, wxb)
            start(g + 3, bufb, wxb, semb)
            return carry

        lax.fori_loop(0, NG // 2, pair, 0)
        wait(bufa, wxa, sema)
        wait(bufb, wxb, semb)

    kern = pl.kernel(
        body,
        out_type=jax.ShapeDtypeStruct((NSC, dh), jnp.float32),
        mesh=mesh,
        scratch_types=[
            pltpu.VMEM((RT * SKS,), jnp.int32),
            pltpu.VMEM((EDG * 16,), jnp.float32),
            pltpu.VMEM((EDG * 16,), jnp.float32),
            pltpu.VMEM((EDG, dh), jnp.float32),
            pltpu.VMEM((EDG, dh), jnp.float32),
            pltpu.VMEM((GN, dh), jnp.float32),
            pltpu.VMEM((dh,), jnp.float32),
            pltpu.SemaphoreType.DMA,
            pltpu.SemaphoreType.DMA,
        ],
    )
    wexp = jnp.broadcast_to(wf[:, None], (NSC * SKS, 16)).reshape(-1)
    return kern(hw, idxf, wexp, bias)


def kernel(x, W1, b1, W2, b2, alpha, gW1, gb1, gW2, gb2):
    x = x.astype(jnp.float32)
    emb, hw1 = _adapter_call(x, W1, b1, W2, b2, alpha, gW1)
    embp = jnp.pad(emb, ((0, NCP - N), (0, 0)))
    idx, w, dinv8 = _simtopk_call(emb, embp)

    dinv = dinv8[:, 0]
    idxk = idx[:, :SKS]
    wk = w[:, :SKS] * dinv[idxk]
    idxf = jnp.pad(idxk, ((0, NSC - N), (0, 0))).reshape(-1)
    wf = jnp.pad(wk, ((0, NSC - N), (0, 0))).reshape(-1)

    msum1 = _sc_mp_call(hw1, idxf, wf, jnp.zeros((H,), jnp.float32), H)[:N]
    cp = 128
    gW2p = jnp.pad(gW2, ((0, 0), (0, cp - C)))
    hw2 = _mlp2_call(msum1, gb1, gW2p, cp)
    gb2p = jnp.pad(gb2, (0, cp - C))
    msum2 = _sc_mp_call(hw2, idxf, wf, gb2p, cp)
    x_graph = msum2[:N, :C]
    return (emb, x_graph)


# speedup vs baseline: 1.2922x; 1.2922x over previous
import functools

import jax
import jax.numpy as jnp
from jax import lax
from jax.experimental import pallas as pl
from jax.experimental.pallas import tpu as pltpu
from jax.experimental.pallas import tpu_sc as plsc

N = 10000
D = 512
BOT = 64
H = 256
C = 46
K = 10

BR = 1000
BC = 2048
NCP = 10240
NEG = -1e30
BIGF = 1e9
KS = 16


def _adapter_body(alpha_ref, x_ref, w1_ref, b1_ref, w2_ref, b2_ref,
                  emb_ref):
    x = x_ref[...]
    t = jnp.maximum(jnp.dot(x, w1_ref[...], preferred_element_type=jnp.float32)
                    + b1_ref[...], 0.0)
    a2 = jnp.dot(t, w2_ref[...], preferred_element_type=jnp.float32) + b2_ref[...]
    alpha = alpha_ref[0]
    ad = alpha * (a2 + x) + (1.0 - alpha) * x
    nrm = jnp.maximum(jnp.sqrt(jnp.sum(ad * ad, axis=1, keepdims=True)), 1e-12)
    emb_ref[...] = ad / nrm


def _adapter_call(x, W1, b1, W2, b2, alpha):
    grid = (N // BR,)
    return pl.pallas_call(
        _adapter_body,
        grid_spec=pltpu.PrefetchScalarGridSpec(
            num_scalar_prefetch=1,
            grid=grid,
            in_specs=[
                pl.BlockSpec((BR, D), lambda i, a: (i, 0)),
                pl.BlockSpec((D, BOT), lambda i, a: (0, 0)),
                pl.BlockSpec((BOT,), lambda i, a: (0,)),
                pl.BlockSpec((BOT, D), lambda i, a: (0, 0)),
                pl.BlockSpec((D,), lambda i, a: (0,)),
            ],
            out_specs=pl.BlockSpec((BR, D), lambda i, a: (i, 0)),
        ),
        out_shape=jax.ShapeDtypeStruct((N, D), jnp.float32),
        compiler_params=pltpu.CompilerParams(
            dimension_semantics=("parallel",)),
    )(jnp.reshape(alpha, (1,)), x, W1, b1, W2, b2)


def _hw1_body(emb_ref, dinv_ref, gw1_ref, hw1_ref):
    es = emb_ref[...] * dinv_ref[:, :1]
    hw1_ref[...] = jnp.dot(es, gw1_ref[...], preferred_element_type=jnp.float32)


def _hw1_call(emb, dinv8, gW1):
    grid = (N // BR,)
    return pl.pallas_call(
        _hw1_body,
        grid=grid,
        in_specs=[
            pl.BlockSpec((BR, D), lambda i: (i, 0)),
            pl.BlockSpec((BR, 8), lambda i: (i, 0)),
            pl.BlockSpec((D, H), lambda i: (0, 0)),
        ],
        out_specs=pl.BlockSpec((BR, H), lambda i: (i, 0)),
        out_shape=jax.ShapeDtypeStruct((N, H), jnp.float32),
        compiler_params=pltpu.CompilerParams(
            dimension_semantics=("parallel",)),
    )(emb, dinv8, gW1)


def _topk_extract(s, gidx, k):
    vs, ids = [], []
    for _ in range(k):
        m = jnp.max(s, axis=1, keepdims=True)
        cand = jnp.where(s >= m, gidx, BIGF)
        am = jnp.min(cand, axis=1, keepdims=True)
        vs.append(m)
        ids.append(am)
        s = jnp.where(cand == am, NEG, s)
    return jnp.concatenate(vs, axis=1), jnp.concatenate(ids, axis=1)


def _simtopk_body(rows_ref, cols_ref, oidx_ref, ow_ref, odinv_ref,
                  sv_ref, si_ref):
    i = pl.program_id(0)
    j = pl.program_id(1)
    nj = pl.num_programs(1)

    s = lax.dot_general(rows_ref[...], cols_ref[...],
                        (((1,), (1,)), ((), ())),
                        preferred_element_type=jnp.float32)
    gcol = (lax.broadcasted_iota(jnp.int32, (BR, BC), 1).astype(jnp.float32)
            + (j * BC).astype(jnp.float32))
    grow = (lax.broadcasted_iota(jnp.int32, (BR, 1), 0).astype(jnp.float32)
            + (i * BR).astype(jnp.float32))
    s = jnp.where(gcol == grow, NEG, s)
    s = jnp.where(gcol >= N, NEG, s)

    tv, ti = _topk_extract(s, gcol, K)

    @pl.when(j == 0)
    def _init():
        sv_ref[...] = jnp.full((BR, KS), NEG, jnp.float32)
        si_ref[...] = jnp.full((BR, KS), -1.0, jnp.float32)

    mv, mi = _topk_extract(jnp.concatenate([sv_ref[...], tv], axis=1),
                           jnp.concatenate([si_ref[...], ti], axis=1), K)
    pad_v = jnp.full((BR, KS - K), NEG, jnp.float32)
    pad_i = jnp.full((BR, KS - K), -1.0, jnp.float32)
    sv_ref[...] = jnp.concatenate([mv, pad_v], axis=1)
    si_ref[...] = jnp.concatenate([mi, pad_i], axis=1)

    @pl.when(j == nj - 1)
    def _emit():
        deg = 1.0 + jnp.sum(mv, axis=1, keepdims=True)
        dinv = jnp.where(deg > 0, lax.rsqrt(jnp.abs(deg) + 1e-30), 0.0)
        growb = jnp.broadcast_to(grow, (BR, KS - K))
        oidx_ref[...] = jnp.concatenate([mi, growb], axis=1).astype(jnp.int32)
        ow_ref[...] = jnp.concatenate(
            [dinv * mv, dinv, jnp.zeros((BR, KS - K - 1), jnp.float32)], axis=1)
        odinv_ref[...] = jnp.broadcast_to(dinv, (BR, 8))


def _simtopk_call(emb, embp):
    grid = (N // BR, NCP // BC)
    return pl.pallas_call(
        _simtopk_body,
        grid=grid,
        in_specs=[
            pl.BlockSpec((BR, D), lambda i, j: (i, 0)),
            pl.BlockSpec((BC, D), lambda i, j: (j, 0)),
        ],
        out_specs=[
            pl.BlockSpec((BR, KS), lambda i, j: (i, 0)),
            pl.BlockSpec((BR, KS), lambda i, j: (i, 0)),
            pl.BlockSpec((BR, 8), lambda i, j: (i, 0)),
        ],
        out_shape=[
            jax.ShapeDtypeStruct((N, KS), jnp.int32),
            jax.ShapeDtypeStruct((N, KS), jnp.float32),
            jax.ShapeDtypeStruct((N, 8), jnp.float32),
        ],
        scratch_shapes=[
            pltpu.VMEM((BR, KS), jnp.float32),
            pltpu.VMEM((BR, KS), jnp.float32),
        ],
        compiler_params=pltpu.CompilerParams(
            dimension_semantics=("parallel", "arbitrary")),
    )(emb, embp)


def _mlp2_body(msum_ref, gb1_ref, dinv_ref, gw2_ref, hw2_ref):
    h = jnp.maximum(msum_ref[...] + gb1_ref[...], 0.0)
    hw2 = jnp.dot(h, gw2_ref[...], preferred_element_type=jnp.float32)
    hw2_ref[...] = hw2 * dinv_ref[:, :1]


def _mlp2_call(msum1, gb1, dinv8, gW2p, cp):
    grid = (N // BR,)
    return pl.pallas_call(
        _mlp2_body,
        grid=grid,
        in_specs=[
            pl.BlockSpec((BR, H), lambda i: (i, 0)),
            pl.BlockSpec((H,), lambda i: (0,)),
            pl.BlockSpec((BR, 8), lambda i: (i, 0)),
            pl.BlockSpec((H, cp), lambda i: (0, 0)),
        ],
        out_specs=pl.BlockSpec((BR, cp), lambda i: (i, 0)),
        out_shape=jax.ShapeDtypeStruct((N, cp), jnp.float32),
        compiler_params=pltpu.CompilerParams(
            dimension_semantics=("parallel",)),
    )(msum1, gb1, dinv8, gW2p)


NSC = 10240
RT = 320
GN = 8
NG = RT // GN
SKS = 12
EDG = GN * SKS


def _sc_mp_call(hw, idxf, wf, bias, dh):
    nv = dh // 16
    mesh = plsc.VectorSubcoreMesh(core_axis_name="c", subcore_axis_name="s")

    def body(hw_hbm, idx_hbm, wx_hbm, bias_hbm, out_hbm,
             idx_v, wxa, wxb, bufa, bufb, outc, bias_v, sema, semb):
        wid = lax.axis_index("s") * 2 + lax.axis_index("c")
        base = wid * RT
        pltpu.sync_copy(idx_hbm.at[pl.ds(base * SKS, RT * SKS)], idx_v)
        pltpu.sync_copy(bias_hbm, bias_v)

        def start(g, buf, wx, sem):
            gg = g % NG
            pltpu.async_copy(
                hw_hbm.at[idx_v.at[pl.ds(gg * EDG, EDG)]], buf, sem)
            pltpu.async_copy(
                wx_hbm.at[pl.ds((base * SKS + gg * EDG) * 16, EDG * 16)],
                wx, sem)

        def wait(buf, wx, sem):
            pltpu.make_async_copy(
                hw_hbm.at[idx_v.at[pl.ds(0, EDG)]], buf, sem).wait()
            pltpu.make_async_copy(
                wx_hbm.at[pl.ds(0, EDG * 16)], wx, sem).wait()

        def compute(g, buf, wx):
            for n in range(GN):
                acc = [bias_v[pl.ds(v * 16, 16)] for v in range(nv)]
                for e in range(SKS):
                    row = n * SKS + e
                    wv = wx[pl.ds(row * 16, 16)]
                    for v in range(nv):
                        acc[v] = acc[v] + wv * buf[row, pl.ds(v * 16, 16)]
                for v in range(nv):
                    outc[n, pl.ds(v * 16, 16)] = acc[v]
            pltpu.sync_copy(outc, out_hbm.at[pl.ds(base + g * GN, GN)])

        start(0, bufa, wxa, sema)
        start(1, bufb, wxb, semb)

        def pair(p, carry):
            g = p * 2
            wait(bufa, wxa, sema)
            compute(g, bufa, wxa)
            start(g + 2, bufa, wxa, sema)
            wait(bufb, wxb, semb)
            compute(g + 1, bufb, wxb)
            start(g + 3, bufb, wxb, semb)
            return carry

        lax.fori_loop(0, NG // 2, pair, 0)
        wait(bufa, wxa, sema)
        wait(bufb, wxb, semb)

    kern = pl.kernel(
        body,
        out_type=jax.ShapeDtypeStruct((NSC, dh), jnp.float32),
        mesh=mesh,
        scratch_types=[
            pltpu.VMEM((RT * SKS,), jnp.int32),
            pltpu.VMEM((EDG * 16,), jnp.float32),
            pltpu.VMEM((EDG * 16,), jnp.float32),
            pltpu.VMEM((EDG, dh), jnp.float32),
            pltpu.VMEM((EDG, dh), jnp.float32),
            pltpu.VMEM((GN, dh), jnp.float32),
            pltpu.VMEM((dh,), jnp.float32),
            pltpu.SemaphoreType.DMA,
            pltpu.SemaphoreType.DMA,
        ],
    )
    wexp = jnp.broadcast_to(wf[:, None], (NSC * SKS, 16)).reshape(-1)
    return kern(hw, idxf, wexp, bias)


def kernel(x, W1, b1, W2, b2, alpha, gW1, gb1, gW2, gb2):
    x = x.astype(jnp.float32)
    emb = _adapter_call(x, W1, b1, W2, b2, alpha)
    embp = jnp.pad(emb, ((0, NCP - N), (0, 0)))
    idx, w, dinv8 = _simtopk_call(emb, embp)

    hw1 = _hw1_call(emb, dinv8, gW1)
    idxf = jnp.pad(idx[:, :SKS], ((0, NSC - N), (0, 0))).reshape(-1)
    wf = jnp.pad(w[:, :SKS], ((0, NSC - N), (0, 0))).reshape(-1)

    msum1 = _sc_mp_call(hw1, idxf, wf, jnp.zeros((H,), jnp.float32), H)[:N]
    cp = 128
    gW2p = jnp.pad(gW2, ((0, 0), (0, cp - C)))
    hw2 = _mlp2_call(msum1, gb1, dinv8, gW2p, cp)
    gb2p = jnp.pad(gb2, (0, cp - C))
    msum2 = _sc_mp_call(hw2, idxf, wf, gb2p, cp)
    x_graph = msum2[:N, :C]
    return (emb, x_graph)
